# baseline (speedup 1.0000x reference)
import functools

import jax
import jax.numpy as jnp
from jax import lax
from jax.experimental import pallas as pl
from jax.experimental.pallas import tpu as pltpu
from jax.experimental.pallas import tpu_sc as plsc

NN = 10000
DD = 128
EE = 160000
SC_NC = 2
SC_NS = 16
SC_NW = SC_NC * SC_NS
ECH = 128
NCHUNK = EE // ECH
KMAX = -(-NCHUNK // SC_NW)
NCHP = 1280
CPT = NCHP // SC_NS
ACC_ROWS = 10240
STRIPE = ACC_ROWS // SC_NS
NPAD = 10240
BI = 1024
BJ = 1024
NIB = NPAD // BI
NJB = NPAD // BJ
INV_TEMP = 2.0
_LOG2E = 1.4426950408889634
_LN2 = 0.6931471805599453


def _dinv_block(degp_ref, i, B):
    deg = jnp.sum(degp_ref[:, pl.ds(i * B, B)], axis=0) + 1.0
    return lax.rsqrt(deg)[:, None]


def _k1_body(x_ref, w_ref, degp_ref, hs_ref):
    i = pl.program_id(0)
    dinv = _dinv_block(degp_ref, i, x_ref.shape[0])
    h = jnp.dot(x_ref[...], w_ref[...], preferred_element_type=jnp.float32)
    hs_ref[...] = h * dinv


def _k1(x, w, degp):
    B = 2048
    grid = (NPAD // B,)
    return pl.pallas_call(
        _k1_body,
        grid=grid,
        in_specs=[
            pl.BlockSpec((B, DD), lambda i: (i, 0)),
            pl.BlockSpec((DD, DD), lambda i: (0, 0)),
            pl.BlockSpec((SC_NS, NPAD), lambda i: (0, 0)),
        ],
        out_specs=pl.BlockSpec((B, DD), lambda i: (i, 0)),
        out_shape=jax.ShapeDtypeStruct((NPAD, DD), jnp.float32),
    )(x, w, degp)


def _k5_body(acc_ref, hs_ref, degp_ref, b_ref, w1_ref, b1_ref,
             w2_ref, b2_ref, a_ref):
    i = pl.program_id(0)
    dinv = _dinv_block(degp_ref, i, acc_ref.shape[0])
    g = dinv * (acc_ref[...] + hs_ref[...]) + b_ref[...]
    t = jnp.dot(g, w1_ref[...], preferred_element_type=jnp.float32) + b1_ref[...]
    z = jnp.where(t > 0.0, t, jnp.exp(jnp.minimum(t, 0.0)) - 1.0)
    z2 = jnp.dot(z, w2_ref[...], preferred_element_type=jnp.float32) + b2_ref[...]
    nrm = jnp.sqrt(jnp.sum(z2 * z2, axis=1, keepdims=True))
    a_ref[...] = z2 / jnp.maximum(nrm, 1e-12)


def _k5(acc, hs, degp, b, w1, b1, w2, b2):
    B = 2048
    grid = (NPAD // B,)
    row = lambda i: (i, 0)
    full = lambda i: (0, 0)
    return pl.pallas_call(
        _k5_body,
        grid=grid,
        in_specs=[
            pl.BlockSpec((B, DD), row),
            pl.BlockSpec((B, DD), row),
            pl.BlockSpec((SC_NS, NPAD), full),
            pl.BlockSpec((1, DD), full),
            pl.BlockSpec((DD, DD), full),
            pl.BlockSpec((1, DD), full),
            pl.BlockSpec((DD, DD), full),
            pl.BlockSpec((1, DD), full),
        ],
        out_specs=pl.BlockSpec((B, DD), row),
        out_shape=jax.ShapeDtypeStruct((NPAD, DD), jnp.float32),
    )(acc, hs, degp, b, w1, b1, w2, b2)


def _dott(x, y):
    return lax.dot_general(x, y, (((1,), (1,)), ((), ())),
                           preferred_element_type=jnp.float32)


def _k6_body(aI_ref, bI_ref, aJ_ref, bJ_ref,
             r11c_ref, r12c_ref, r22c_ref, d11_ref, d12_ref, d22_ref,
             r11l_ref, r22l_ref, c12l_ref):
    i = pl.program_id(0)
    j = pl.program_id(1)
    aI = aI_ref[...]
    bI = bI_ref[...]
    aJ = aJ_ref[...]
    bJ = bJ_ref[...]
    s12 = _dott(aI, bJ)
    e12 = jnp.exp2(s12)
    jm1 = ((lax.broadcasted_iota(jnp.int32, (1, BJ), 1) + j * BJ) < NN
           ).astype(jnp.float32)
    imcol = ((lax.broadcasted_iota(jnp.int32, (BI, 1), 0) + i * BI) < NN
             ).astype(jnp.float32)

    @pl.when(j == 0)
    def _():
        r11c_ref[...] = jnp.zeros_like(r11c_ref)
        r12c_ref[...] = jnp.zeros_like(r12c_ref)
        r22c_ref[...] = jnp.zeros_like(r22c_ref)
        d11_ref[...] = jnp.zeros_like(d11_ref)
        d12_ref[...] = jnp.zeros_like(d12_ref)
        d22_ref[...] = jnp.zeros_like(d22_ref)

    @pl.when(jnp.logical_and(i == 0, j == 0))
    def _():
        r11l_ref[...] = jnp.zeros_like(r11l_ref)
        r22l_ref[...] = jnp.zeros_like(r22l_ref)
        c12l_ref[...] = jnp.zeros_like(c12l_ref)

    r12c_ref[...] += _dott(e12, jm1)
    c12l_ref[pl.ds(j, 1), :] += jnp.sum(e12 * imcol, axis=0).reshape(1, BJ)

    @pl.when(i <= j)
    def _():
        e11 = jnp.exp2(_dott(aI, aJ))
        e22 = jnp.exp2(_dott(bI, bJ))
        r11c_ref[...] += _dott(e11, jm1)
        r22c_ref[...] += _dott(e22, jm1)

        @pl.when(i < j)
        def _():
            r11l_ref[pl.ds(j, 1), :] += jnp.sum(e11, axis=0).reshape(1, BJ)
            r22l_ref[pl.ds(j, 1), :] += jnp.sum(e22, axis=0).reshape(1, BJ)

        @pl.when(i == j)
        def _():
            eye = (lax.broadcasted_iota(jnp.int32, (BI, BJ), 0)
                   == lax.broadcasted_iota(jnp.int32, (BI, BJ), 1)
                   ).astype(jnp.float32)
            d11_ref[...] += _dott(e11 * eye, jm1)
            d12_ref[...] += _dott(s12 * eye, jm1)
            d22_ref[...] += _dott(e22 * eye, jm1)


def _k6(a1p, a2p, a1s, a2s):
    colspec = pl.BlockSpec((BI, 1), lambda i, j: (i, 0))
    col = jax.ShapeDtypeStruct((NPAD, 1), jnp.float32)
    lanespec = pl.BlockSpec((NIB, BI), lambda i, j: (0, 0))
    lane = jax.ShapeDtypeStruct((NIB, BI), jnp.float32)
    return pl.pallas_call(
        _k6_body,
        grid=(NIB, NJB),
        in_specs=[
            pl.BlockSpec((BI, DD), lambda i, j: (i, 0)),
            pl.BlockSpec((BI, DD), lambda i, j: (i, 0)),
            pl.BlockSpec((BJ, DD), lambda i, j: (j, 0)),
            pl.BlockSpec((BJ, DD), lambda i, j: (j, 0)),
        ],
        out_specs=[colspec, colspec, colspec, colspec, colspec, colspec,
                   lanespec, lanespec, lanespec],
        out_shape=[col, col, col, col, col, col, lane, lane, lane],
    )(a1p, a2p, a1s, a2s)


def _k7_body(r11c_ref, r11l_ref, r12_ref, d11_ref, d12_ref,
             r22c_ref, r22l_ref, c12_ref, d22_ref, out_ref):
    k = pl.program_id(0)
    imc = ((lax.broadcasted_iota(jnp.int32, (BI, 1), 0) + k * BI) < NN
           ).astype(jnp.float32)
    x1 = r11c_ref[...] + r11l_ref[...] + r12_ref[...] - d11_ref[...]
    x2 = r22c_ref[...] + r22l_ref[...] + c12_ref[...] - d22_ref[...]
    p = jnp.sum((0.5 * (jnp.log(x1) + jnp.log(x2))
                 - _LN2 * d12_ref[...]) * imc)

    @pl.when(k == 0)
    def _():
        out_ref[0, 0] = 0.0

    out_ref[0, 0] += p * (1.0 / NN)


def _k7(r11c, r11l, r12, d11, d12, r22c, r22l, c12, d22):
    colspec = pl.BlockSpec((BI, 1), lambda k: (k, 0))
    return pl.pallas_call(
        _k7_body,
        grid=(NIB,),
        in_specs=[colspec] * 9,
        out_specs=pl.BlockSpec(memory_space=pltpu.SMEM),
        out_shape=jax.ShapeDtypeStruct((1, 1), jnp.float32),
    )(r11c, r11l, r12, d11, d12, r22c, r22l, c12, d22)


@functools.lru_cache(maxsize=None)
def _sc_mesh():
    return plsc.VectorSubcoreMesh(core_axis_name="c", subcore_axis_name="s",
                                  num_cores=SC_NC, num_subcores=SC_NS)


def _sc_deg(edge_st):
    return pl.kernel(
        _sc_deg_body,
        out_type=jax.ShapeDtypeStruct((SC_NC, SC_NS, NN), jnp.float32),
        mesh=_sc_mesh(),
        scratch_types=[
            pltpu.VMEM((CPT, ECH), jnp.int32),
            pltpu.VMEM((ACC_ROWS,), jnp.float32),
        ],
        compiler_params=pltpu.CompilerParams(needs_layout_passes=False,
                                             use_tc_tiling_on_sc=False),
    )(edge_st)


def _sc_deg_body(edge_ref, deg_ref, idx_v, degloc):
    c = lax.axis_index("c")
    s = lax.axis_index("s")
    base = s * CPT

    def zb(k, carry):
        degloc[pl.ds(k * 16, 16)] = jnp.zeros((16,), jnp.float32)
        return carry

    lax.fori_loop(0, ACC_ROWS // 16, zb, None)
    pltpu.sync_copy(edge_ref.at[c, 1, pl.ds(base, CPT)], idx_v)
    ones = jnp.ones((16,), jnp.float32)

    def chunk(kl, carry):
        g = base + kl

        @pl.when(g < NCHUNK)
        def _():
            for t in range(ECH // 16):
                idx = idx_v[kl, pl.ds(t * 16, 16)]
                plsc.addupdate_scatter(degloc, [idx], ones)

        return carry

    lax.fori_loop(0, CPT, chunk, None)
    pltpu.sync_copy(degloc.at[pl.ds(0, NN)], deg_ref.at[c, s])


def _sc_scatter(hs_st, edge_st):
    return pl.kernel(
        _sc_scatter_body,
        out_type=jax.ShapeDtypeStruct((SC_NC, ACC_ROWS, DD), jnp.float32),
        mesh=_sc_mesh(),
        scratch_types=[
            pltpu.VMEM((CPT // 2, ECH), jnp.int32),
            pltpu.VMEM((CPT // 2, ECH), jnp.int32),
            pltpu.VMEM((ECH, DD), jnp.float32),
            pltpu.VMEM((ECH, DD), jnp.float32),
            pltpu.VMEM_SHARED((ACC_ROWS, DD), jnp.float32),
            pltpu.SemaphoreType.DMA,
            pltpu.SemaphoreType.DMA,
        ],
        compiler_params=pltpu.CompilerParams(needs_layout_passes=False,
                                             use_tc_tiling_on_sc=False),
    )(hs_st, edge_st)


def _sc_scatter_body(hs_ref, edge_ref, acc_ref, src_v, dst_v, rows0, rows1,
                     acc_sh, sem0, sem1):
    c = lax.axis_index("c")
    s = lax.axis_index("s")
    base = s * CPT
    hs_c = hs_ref.at[c]

    def zb(k, carry):
        r = k // (DD // 16)
        t = k % (DD // 16)
        rows0[r, pl.ds(t * 16, 16)] = jnp.zeros((16,), jnp.float32)
        return carry

    lax.fori_loop(0, ECH * (DD // 16), zb, None)
    for sblk in range(STRIPE // ECH):
        pltpu.sync_copy(rows0, acc_sh.at[pl.ds(s * STRIPE + sblk * ECH, ECH)])
    plsc.subcore_barrier()

    bufs = (rows0, rows1)
    sems = (sem0, sem1)
    HC = CPT // 2

    for h in range(2):
        hbase = base + h * HC
        pltpu.sync_copy(edge_ref.at[c, 0, pl.ds(hbase, HC)], src_v)
        pltpu.sync_copy(edge_ref.at[c, 1, pl.ds(hbase, HC)], dst_v)

        @pl.when(hbase < NCHUNK)
        def _():
            pltpu.async_copy(hs_c.at[src_v.at[0]], rows0, sem0)

        def pair(k2, carry):
            for ph in range(2):
                kl = 2 * k2 + ph
                g = hbase + kl
                buf = bufs[ph]
                sem = sems[ph]
                obuf = bufs[1 - ph]
                osem = sems[1 - ph]

                @pl.when(g < NCHUNK)
                def _():
                    pltpu.make_async_copy(hs_c.at[src_v.at[kl]], buf,
                                          sem).wait()

                @pl.when(jnp.logical_and(g + 1 < NCHUNK, kl + 1 < HC))
                def _():
                    pltpu.async_copy(hs_c.at[src_v.at[kl + 1]], obuf, osem)

                @pl.when(g < NCHUNK)
                def _():
                    pltpu.sync_copy(buf, acc_sh.at[dst_v.at[kl]], add=True)

            return carry

        lax.fori_loop(0, HC // 2, pair, None)
    plsc.subcore_barrier()
    pltpu.sync_copy(acc_sh.at[pl.ds(s * STRIPE, STRIPE)],
                    acc_ref.at[c, pl.ds(s * STRIPE, STRIPE)])


def kernel(edge1, edge2, feat1, feat2, W_gcn, b_gcn, fc1_W, fc1_b, fc2_W,
           fc2_b):
    f1p = jnp.pad(feat1, ((0, NPAD - NN), (0, 0)))
    f2p = jnp.pad(feat2, ((0, NPAD - NN), (0, 0)))
    e1r = jnp.pad(edge1.reshape(2, NCHUNK, ECH),
                  ((0, 0), (0, NCHP - NCHUNK), (0, 0)))
    e2r = jnp.pad(edge2.reshape(2, NCHUNK, ECH),
                  ((0, 0), (0, NCHP - NCHUNK), (0, 0)))
    edge_st = jnp.stack([e1r, e2r])
    degp = _sc_deg(edge_st)
    degp1 = jnp.pad(degp[0], ((0, 0), (0, NPAD - NN)))
    degp2 = jnp.pad(degp[1], ((0, 0), (0, NPAD - NN)))
    hs1 = _k1(f1p, W_gcn, degp1)
    hs2 = _k1(f2p, W_gcn, degp2)
    accp = _sc_scatter(jnp.stack([hs1, hs2]), edge_st)
    b2d = b_gcn.reshape(1, DD)
    b1d = fc1_b.reshape(1, DD)
    b2d2 = fc2_b.reshape(1, DD)
    a1p = _k5(accp[0], hs1, degp1, b2d, fc1_W, b1d, fc2_W, b2d2)
    a2p = _k5(accp[1], hs2, degp2, b2d, fc1_W, b1d, fc2_W, b2d2)
    r11c, r12c, r22c, d11, d12, d22, r11l, r22l, c12l = _k6(
        a1p.astype(jnp.bfloat16), a2p.astype(jnp.bfloat16),
        (a1p * (INV_TEMP * _LOG2E)).astype(jnp.bfloat16),
        (a2p * (INV_TEMP * _LOG2E)).astype(jnp.bfloat16))
    loss = _k7(r11c, r11l.reshape(NPAD, 1), r12c, d11, d12,
               r22c, r22l.reshape(NPAD, 1), c12l.reshape(NPAD, 1), d22)
    return loss[0, 0]
